# baseline (speedup 1.0000x reference)
import jax
import jax.numpy as jnp
from jax import lax
from jax.experimental import pallas as pl
from jax.experimental.pallas import tpu as pltpu, tpu_sc as plsc

B = 16384
D = 64
NC, NS, L = 2, 16, 16
NW = NC * NS
RPW = B // NW
SUB = 64
NSUB = RPW // SUB
BLK = 128
NBLK = RPW // BLK


def _body(h_hbm, r_hbm, t_hbm, n_hbm, ent_hbm, rel_hbm, score_hbm, neg_hbm,
          hs, rs, ts, ns,
          hi2, ri2, ti2, ni2, hof, rof, tof, nof,
          hb0, rb0, tb0, nb0, hb1, rb1, tb1, nb1,
          so0, no0, so1, no1,
          isem, gsem0, gsem1, osem0, osem1):
    wid = lax.axis_index("s") * NC + lax.axis_index("c")
    base_w = wid * RPW
    col0 = base_w

    cps = [pltpu.async_copy(h_hbm.at[pl.ds(base_w, RPW)], hs, isem),
           pltpu.async_copy(r_hbm.at[pl.ds(base_w, RPW)], rs, isem),
           pltpu.async_copy(t_hbm.at[pl.ds(base_w, RPW)], ts, isem),
           pltpu.async_copy(n_hbm.at[pl.ds(base_w, RPW)], ns, isem)]
    for c in cps:
        c.wait()

    for src, idx2, off in ((hs, hi2, hof), (rs, ri2, rof),
                           (ts, ti2, tof), (ns, ni2, nof)):
        for v in range(RPW // L):
            val = src[pl.ds(v * L, L)]
            sub, lo = (v * L) // SUB, (v * L) % SUB
            idx2[sub, pl.ds(lo, L)] = lax.shift_right_logical(val, 1)
            off[sub, pl.ds(lo, L)] = lax.shift_left(
                lax.bitwise_and(val, 1), 6)

    gsets = ((hb0, rb0, tb0, nb0, gsem0), (hb1, rb1, tb1, nb1, gsem1))
    osets = ((so0, no0, osem0), (so1, no1, osem1))

    def start_gathers(k):
        hb, rb, tb, nb, gsem = gsets[k % 2]
        return [pltpu.async_copy(ent_hbm.at[hi2.at[k]], hb, gsem),
                pltpu.async_copy(rel_hbm.at[ri2.at[k]], rb, gsem),
                pltpu.async_copy(ent_hbm.at[ti2.at[k]], tb, gsem),
                pltpu.async_copy(ent_hbm.at[ni2.at[k]], nb, gsem)]

    iota = lax.iota(jnp.int32, L)
    pend_g = {0: start_gathers(0)}
    pend_o = {}
    for k in range(NSUB):
        hb, rb, tb, nb, _ = gsets[k % 2]
        blk = k // 2
        outS, outN, osem = osets[blk % 2]
        if k + 1 < NSUB:
            pend_g[k + 1] = start_gathers(k + 1)
        if k % 2 == 0:
            for c in pend_o.pop(blk - 2, ()):
                c.wait()
        for c in pend_g.pop(k):
            c.wait()

        kv = jnp.full((L,), k, jnp.int32)
        half = (k % 2) * SUB

        @plsc.parallel_loop(0, SUB)
        def _compute(i):
            iv = i + jnp.zeros((L,), jnp.int32)
            bh = plsc.load_gather(hof, [kv, iv])
            br = plsc.load_gather(rof, [kv, iv])
            bt = plsc.load_gather(tof, [kv, iv])
            bn = plsc.load_gather(nof, [kv, iv])
            co = iv + half
            for j in range(D // L):
                cj = iota + (j * L)
                vh = plsc.load_gather(hb, [iv, bh + cj])
                vr = plsc.load_gather(rb, [iv, br + cj])
                vt = plsc.load_gather(tb, [iv, bt + cj])
                vn = plsc.load_gather(nb, [iv, bn + cj])
                s = vh + vr
                plsc.store_scatter(outS, [cj, co], s - vt)
                plsc.store_scatter(outN, [cj, co], s - vn)

        if k % 2 == 1:
            cb = col0 + blk * BLK
            pend_o[blk] = [
                pltpu.async_copy(outS, score_hbm.at[:, pl.ds(cb, BLK)], osem),
                pltpu.async_copy(outN, neg_hbm.at[:, pl.ds(cb, BLK)], osem)]
    for b in sorted(pend_o):
        for c in pend_o[b]:
            c.wait()


def kernel(h, r, t, neg_idx, entity_table, relation_table):
    ne, nr = entity_table.shape[0], relation_table.shape[0]
    et2 = jnp.reshape(entity_table, (ne // 2, 2 * D))
    rt2 = jnp.reshape(relation_table, (nr // 2, 2 * D))
    mesh = plsc.VectorSubcoreMesh(
        core_axis_name="c", subcore_axis_name="s",
        num_cores=NC, num_subcores=NS)
    f = pl.kernel(
        _body,
        out_type=(jax.ShapeDtypeStruct((D, B), jnp.float32),
                  jax.ShapeDtypeStruct((D, B), jnp.float32)),
        mesh=mesh,
        compiler_params=pltpu.CompilerParams(needs_layout_passes=False),
        scratch_types=[
            pltpu.VMEM((RPW,), jnp.int32),
            pltpu.VMEM((RPW,), jnp.int32),
            pltpu.VMEM((RPW,), jnp.int32),
            pltpu.VMEM((RPW,), jnp.int32),
            pltpu.VMEM((NSUB, SUB), jnp.int32),
            pltpu.VMEM((NSUB, SUB), jnp.int32),
            pltpu.VMEM((NSUB, SUB), jnp.int32),
            pltpu.VMEM((NSUB, SUB), jnp.int32),
            pltpu.VMEM((NSUB, SUB), jnp.int32),
            pltpu.VMEM((NSUB, SUB), jnp.int32),
            pltpu.VMEM((NSUB, SUB), jnp.int32),
            pltpu.VMEM((NSUB, SUB), jnp.int32),
            pltpu.VMEM((SUB, 2 * D), jnp.float32),
            pltpu.VMEM((SUB, 2 * D), jnp.float32),
            pltpu.VMEM((SUB, 2 * D), jnp.float32),
            pltpu.VMEM((SUB, 2 * D), jnp.float32),
            pltpu.VMEM((SUB, 2 * D), jnp.float32),
            pltpu.VMEM((SUB, 2 * D), jnp.float32),
            pltpu.VMEM((SUB, 2 * D), jnp.float32),
            pltpu.VMEM((SUB, 2 * D), jnp.float32),
            pltpu.VMEM((D, BLK), jnp.float32),
            pltpu.VMEM((D, BLK), jnp.float32),
            pltpu.VMEM((D, BLK), jnp.float32),
            pltpu.VMEM((D, BLK), jnp.float32),
            pltpu.SemaphoreType.DMA,
            pltpu.SemaphoreType.DMA,
            pltpu.SemaphoreType.DMA,
            pltpu.SemaphoreType.DMA,
            pltpu.SemaphoreType.DMA,
        ],
    )
    score, neg = f(h.astype(jnp.int32), r.astype(jnp.int32),
                   t.astype(jnp.int32), neg_idx.astype(jnp.int32),
                   et2, rt2)
    return score.T[:, None, :], neg.T[:, None, :]
